# Initial kernel scaffold; baseline (speedup 1.0000x reference)
import jax
import jax.numpy as jnp
from jax.experimental import pallas as pl


def kernel(x, text_duration, train):
    raise NotImplementedError("write your pallas kernel here")


# speedup vs baseline: 8.6381x; 8.6381x over previous
import math

import jax
import jax.numpy as jnp
from jax.experimental import pallas as pl

_T, _B, _C, _W = 2048, 8, 512, 64
_PE_ROWS = 32
_TBLK = 256


def _pe_table():
    pos = jnp.arange(_PE_ROWS, dtype=jnp.float32)[:, None]
    div = jnp.exp(
        jnp.arange(0, _C, 2, dtype=jnp.float32) * (-math.log(10000.0) / _C)
    )
    ang = pos * div
    pe = jnp.stack([jnp.sin(ang), jnp.cos(ang)], axis=-1).reshape(_PE_ROWS, _C)
    return pe


def _body(dur_ref, x_ref, pe_ref, o_ref):
    i = pl.program_id(0)
    dur = dur_ref[...].astype(jnp.float32)
    tri = (
        jax.lax.broadcasted_iota(jnp.int32, (_W, _W), 0)
        <= jax.lax.broadcasted_iota(jnp.int32, (_W, _W), 1)
    ).astype(jnp.float32)
    csum = jnp.dot(dur, tri, preferred_element_type=jnp.float32).astype(
        jnp.int32
    )

    t3 = jax.lax.broadcasted_iota(jnp.int32, (_TBLK, _B, _W), 0) + i * _TBLK
    le = csum[None, :, :] <= t3
    start = jnp.max(jnp.where(le, csum[None, :, :], 0), axis=2)

    t2 = jax.lax.broadcasted_iota(jnp.int32, (_TBLK, _B), 0) + i * _TBLK
    total = csum[:, _W - 1]
    mask = t2 < total[None, :]
    within = jnp.where(mask, t2 - start, _PE_ROWS)

    oh = (
        within[:, :, None]
        == jax.lax.broadcasted_iota(jnp.int32, (_TBLK, _B, _PE_ROWS), 2)
    )
    ohf = oh.astype(jnp.float32).reshape(_TBLK * _B, _PE_ROWS)
    add = jnp.dot(ohf, pe_ref[...], preferred_element_type=jnp.float32)
    o_ref[...] = x_ref[...] + add.reshape(_TBLK, _B, _C)


def kernel(x, text_duration, train):
    del train
    pe = _pe_table()
    grid = _T // _TBLK
    out = pl.pallas_call(
        _body,
        grid=(grid,),
        in_specs=[
            pl.BlockSpec((_B, _W), lambda i: (0, 0)),
            pl.BlockSpec((_TBLK, _B, _C), lambda i: (i, 0, 0)),
            pl.BlockSpec((_PE_ROWS, _C), lambda i: (0, 0)),
        ],
        out_specs=pl.BlockSpec((_TBLK, _B, _C), lambda i: (i, 0, 0)),
        out_shape=jax.ShapeDtypeStruct((_T, _B, _C), jnp.float32),
    )(text_duration, x, pe)
    return out
